# baseline (speedup 1.0000x reference)
import functools
import math

import jax
import jax.numpy as jnp
from jax import lax
from jax.experimental import pallas as pl
from jax.experimental.pallas import tpu as pltpu
from jax.experimental.pallas import tpu_sc as plsc

T = 2048
D = 1024
E = 8
HS = 2 * D
HR = 3 * D
TAU = 1.5
BLK = 128
NB = 40
S = NB * BLK
BH = 1024
NBH = HR // BH
BH_S = 1024
NBH_S = HS // BH_S

_F32 = jnp.float32
_PREC = lax.Precision.DEFAULT
_BF16 = jnp.bfloat16


def _gelu(v):
    return 0.5 * v * (1.0 + lax.erf(v * (1.0 / math.sqrt(2.0))))


def _router_body(lg_ref, idx_ref, w_ref):
    logits = lg_ref[...]
    col = lax.broadcasted_iota(jnp.int32, (T, E), 1)
    v1 = jnp.max(logits, axis=-1, keepdims=True)
    i1 = jnp.min(jnp.where(logits == v1, col, E), axis=-1, keepdims=True)
    masked = jnp.where(col == i1, -jnp.inf, logits)
    v2 = jnp.max(masked, axis=-1, keepdims=True)
    i2 = jnp.min(jnp.where(masked == v2, col, E), axis=-1, keepdims=True)
    w1 = 1.0 / (1.0 + jnp.exp(v2 - v1))
    idx_ref[:, 0:1] = i1
    idx_ref[:, 1:2] = i2
    w_ref[:, 0:1] = w1
    w_ref[:, 1:2] = 1.0 - w1


def _router(xm, r_w, r_b):
    logits = (xm @ r_w + r_b) / max(TAU, 1e-6)
    return pl.pallas_call(
        _router_body,
        out_shape=(jax.ShapeDtypeStruct((T, 2), jnp.int32),
                   jax.ShapeDtypeStruct((T, 2), _F32)),
    )(logits)


BT_S = 256
NT_S = T // BT_S


def _shared_body(x_ref, w1a_ref, w1g_ref, b1a_ref, b1g_ref, w2_ref, b2_ref,
                 out_ref):
    x = x_ref[...]
    a = jnp.dot(x, w1a_ref[...], precision=_PREC,
                preferred_element_type=_F32) + b1a_ref[...]
    g = jnp.dot(x, w1g_ref[...], precision=_PREC,
                preferred_element_type=_F32) + b1g_ref[...]
    out_ref[...] = jnp.dot(a * _gelu(g), w2_ref[...], precision=_PREC,
                           preferred_element_type=_F32) + b2_ref[...]


def _shared_mlp(xm, sh_w1, sh_b1, sh_w2, sh_b2):
    b1 = sh_b1.reshape(1, 2 * HS)
    return pl.pallas_call(
        _shared_body,
        grid=(NT_S,),
        in_specs=[
            pl.BlockSpec((BT_S, D), lambda t: (t, 0)),
            pl.BlockSpec((D, HS), lambda t: (0, 0)),
            pl.BlockSpec((D, HS), lambda t: (0, 1)),
            pl.BlockSpec((1, HS), lambda t: (0, 0)),
            pl.BlockSpec((1, HS), lambda t: (0, 1)),
            pl.BlockSpec((HS, D), lambda t: (0, 0)),
            pl.BlockSpec((1, D), lambda t: (0, 0)),
        ],
        out_specs=pl.BlockSpec((BT_S, D), lambda t: (t, 0)),
        out_shape=jax.ShapeDtypeStruct((T, D), _F32),
    )(xm, sh_w1, sh_w1, b1, b1, sh_w2, sh_b2.reshape(1, D))


def _gmm_body(be_ref, xg_ref, w1a_ref, w1g_ref, b1a_ref, b1g_ref, w2_ref,
              b2_ref, sw_ref, out_ref):
    bh = pl.program_id(0)
    b = pl.program_id(1)
    x = xg_ref[...]
    a = jnp.dot(x, w1a_ref[0], precision=_PREC,
                preferred_element_type=_F32) + b1a_ref[0]
    g = jnp.dot(x, w1g_ref[0], precision=_PREC,
                preferred_element_type=_F32) + b1g_ref[0]
    contrib = jnp.dot(a * _gelu(g), w2_ref[0], precision=_PREC,
                      preferred_element_type=_F32)
    rows = pl.ds(b * BLK, BLK)

    @pl.when(bh == 0)
    def _():
        out_ref[rows, :] = contrib

    @pl.when(bh == 1)
    def _():
        out_ref[rows, :] += contrib

    @pl.when(bh == NBH - 1)
    def _():
        out_ref[rows, :] = ((out_ref[rows, :] + contrib + b2_ref[0]) *
                            sw_ref[0, 0][:, None])


def _gmm(xg, e_w1, e_b1, e_w2, e_b2, slot_w, blk_expert):
    grid_spec = pltpu.PrefetchScalarGridSpec(
        num_scalar_prefetch=1,
        grid=(NBH, NB),
        in_specs=[
            pl.BlockSpec((BLK, D), lambda bh, b, be: (b, 0)),
            pl.BlockSpec((1, D, BH), lambda bh, b, be: (be[b], 0, bh)),
            pl.BlockSpec((1, D, BH), lambda bh, b, be: (be[b], 0, bh + NBH)),
            pl.BlockSpec((1, 1, BH), lambda bh, b, be: (be[b], 0, bh)),
            pl.BlockSpec((1, 1, BH), lambda bh, b, be: (be[b], 0, bh + NBH)),
            pl.BlockSpec((1, BH, D), lambda bh, b, be: (be[b], bh, 0)),
            pl.BlockSpec((1, 1, D), lambda bh, b, be: (be[b], 0, 0)),
            pl.BlockSpec((1, 1, BLK), lambda bh, b, be: (b, 0, 0)),
        ],
        out_specs=pl.BlockSpec((S, D), lambda bh, b, be: (0, 0)),
    )
    return pl.pallas_call(
        _gmm_body,
        grid_spec=grid_spec,
        out_shape=jax.ShapeDtypeStruct((S, D), _F32),
    )(blk_expert, xg, e_w1, e_w1, e_b1.reshape(E, 1, 2 * HR),
      e_b1.reshape(E, 1, 2 * HR), e_w2, e_b2.reshape(E, 1, D),
      slot_w.reshape(NB, 1, BLK))


_NC, _NS, _L = 2, 16, 16
_NW = _NC * _NS
_NSTEP = (2 * T) // _L
_SINIT = S // _L
_NBPAD = 48

_SC_MESH = plsc.VectorSubcoreMesh(core_axis_name="c", subcore_axis_name="s")


def _wid():
    return lax.axis_index("s") * _NC + lax.axis_index("c")


def _vgather(vec, idx):
    return vec.at[idx].get(mode="promise_in_bounds")


def _dispatch_body(ti_hbm, tw_hbm, st_hbm, sw_hbm, be_hbm, sl_hbm,
                   ids_v, w_v, st_v, sw_v, be_v, sl_v):
    @pl.when(_wid() == 0)
    def _():
        pltpu.sync_copy(ti_hbm, ids_v)
        pltpu.sync_copy(tw_hbm, w_v)
        lanes = lax.iota(jnp.int32, _L)

        def count_step(i, counts):
            idv = ids_v[pl.ds(i * _L, _L)]
            for e in range(E):
                c = jnp.sum(jnp.where(idv == e, 1, 0))
                counts = counts + jnp.where(lanes == e, c, 0)
            return counts

        counts = lax.fori_loop(0, _NSTEP, count_step,
                               jnp.zeros((_L,), jnp.int32))
        blocks = (counts + (BLK - 1)) >> 7
        cumb = plsc.cumsum(blocks)
        row_off = (cumb - blocks) * BLK

        for ch in range(_NBPAD // _L):
            jb = lanes + ch * _L
            be = jnp.zeros((_L,), jnp.int32)
            for e in range(E - 1):
                ce = jnp.sum(jnp.where(lanes == e, cumb, 0))
                be = be + jnp.where(jb >= ce, 1, 0)
            be_v[pl.ds(ch * _L, _L)] = be

        def zero_step(i, carry):
            st_v[pl.ds(i * _L, _L)] = jnp.zeros((_L,), jnp.int32)
            sw_v[pl.ds(i * _L, _L)] = jnp.zeros((_L,), _F32)
            return carry

        lax.fori_loop(0, _SINIT, zero_step, 0)

        def scat_step(i, fill):
            idv = ids_v[pl.ds(i * _L, _L)]
            rank = jnp.zeros((_L,), jnp.int32)
            newfill = fill
            for e in range(E):
                m = idv == e
                mi = jnp.where(m, 1, 0)
                cs = plsc.cumsum(mi)
                rank = jnp.where(m, cs - mi, rank)
                c = jnp.sum(jnp.where(lanes == _L - 1, cs, 0))
                newfill = newfill + jnp.where(lanes == e, c, 0)
            slot = _vgather(row_off + fill, idv) + rank
            tok = (lanes + i * _L) >> 1
            plsc.store_scatter(st_v, [slot], tok)
            plsc.store_scatter(sw_v, [slot], w_v[pl.ds(i * _L, _L)])
            sl_v[pl.ds(i * _L, _L)] = slot
            return newfill

        lax.fori_loop(0, _NSTEP, scat_step, jnp.zeros((_L,), jnp.int32))
        pltpu.sync_copy(st_v, st_hbm)
        pltpu.sync_copy(sw_v, sw_hbm)
        pltpu.sync_copy(be_v, be_hbm)
        pltpu.sync_copy(sl_v, sl_hbm)


def _sc_dispatch(ti_flat, tw_flat):
    return pl.kernel(
        _dispatch_body,
        out_type=(jax.ShapeDtypeStruct((S,), jnp.int32),
                  jax.ShapeDtypeStruct((S,), _F32),
                  jax.ShapeDtypeStruct((_NBPAD,), jnp.int32),
                  jax.ShapeDtypeStruct((2 * T,), jnp.int32)),
        mesh=_SC_MESH,
        compiler_params=pltpu.CompilerParams(needs_layout_passes=False),
        scratch_types=[pltpu.VMEM((2 * T,), jnp.int32),
                       pltpu.VMEM((2 * T,), _F32),
                       pltpu.VMEM((S,), jnp.int32),
                       pltpu.VMEM((S,), _F32),
                       pltpu.VMEM((_NBPAD,), jnp.int32),
                       pltpu.VMEM((2 * T,), jnp.int32)],
    )(ti_flat, tw_flat)


_RPW = S // _NW
_GCH = 40
_NGCH = _RPW // _GCH


def _gather_body(x_hbm, idx_hbm, out_hbm, i0, i1, i2, i3, r0, r1,
                 gsem, ssem):
    base = _wid() * _RPW
    idx = [i0, i1, i2, i3]
    rows = [r0, r1]
    for ch in range(_NGCH):
        pltpu.sync_copy(idx_hbm.at[pl.ds(base + ch * _GCH, _GCH)], idx[ch])

    def _fire(ch):
        return pltpu.async_copy(x_hbm.at[idx[ch]], rows[ch % 2], gsem)

    def _store(ch):
        return pltpu.async_copy(rows[ch % 2],
                                out_hbm.at[pl.ds(base + ch * _GCH, _GCH)],
                                ssem)

    g = {0: _fire(0)}
    st = {}
    for ch in range(_NGCH):
        if ch + 1 < _NGCH:
            if ch - 1 >= 0:
                st[ch - 1].wait()
            g[ch + 1] = _fire(ch + 1)
        g[ch].wait()
        st[ch] = _store(ch)
    st[_NGCH - 2].wait()
    st[_NGCH - 1].wait()


def _sc_gather(xm, src_token):
    return pl.kernel(
        _gather_body,
        out_type=jax.ShapeDtypeStruct((S, D), _F32),
        mesh=_SC_MESH,
        compiler_params=pltpu.CompilerParams(needs_layout_passes=False),
        scratch_types=[pltpu.VMEM((_GCH,), jnp.int32),
                       pltpu.VMEM((_GCH,), jnp.int32),
                       pltpu.VMEM((_GCH,), jnp.int32),
                       pltpu.VMEM((_GCH,), jnp.int32),
                       pltpu.VMEM((_GCH, D), _F32),
                       pltpu.VMEM((_GCH, D), _F32),
                       pltpu.SemaphoreType.DMA,
                       pltpu.SemaphoreType.DMA],
    )(xm, src_token)


_TPW = T // _NW
_CT = 16
_NCT = _TPW // _CT


def _combine_body(ysh_hbm, routed_hbm, sl_hbm, out_hbm,
                  i0, i1, i2, i3, y0, y1, r0, r1, gsem, ssem):
    tb = _wid() * _TPW
    idx = [i0, i1, i2, i3]
    yb = [y0, y1]
    rb = [r0, r1]
    for ch in range(_NCT):
        pltpu.sync_copy(sl_hbm.at[pl.ds(2 * (tb + ch * _CT), 2 * _CT)],
                        idx[ch])

    def _fire(ch):
        b = ch % 2
        return (pltpu.async_copy(ysh_hbm.at[pl.ds(tb + ch * _CT, _CT)],
                                 yb[b], gsem),
                pltpu.async_copy(routed_hbm.at[idx[ch]], rb[b], gsem))

    def _store(ch):
        return pltpu.async_copy(yb[ch % 2],
                                out_hbm.at[pl.ds(tb + ch * _CT, _CT)], ssem)

    g = {0: _fire(0)}
    st = {}
    for ch in range(_NCT):
        b = ch % 2
        if ch + 1 < _NCT:
            if ch - 1 >= 0:
                st[ch - 1].wait()
            g[ch + 1] = _fire(ch + 1)
        g[ch][0].wait()
        g[ch][1].wait()

        def tok_step(tt, carry):
            def col_step(cc, carry2):
                c0 = cc * _L
                yb[b][tt, pl.ds(c0, _L)] = (yb[b][tt, pl.ds(c0, _L)]
                                            + rb[b][2 * tt, pl.ds(c0, _L)]
                                            + rb[b][2 * tt + 1, pl.ds(c0, _L)])
                return carry2

            lax.fori_loop(0, D // _L, col_step, 0)
            return carry

        lax.fori_loop(0, _CT, tok_step, 0)
        st[ch] = _store(ch)
    st[_NCT - 2].wait()
    st[_NCT - 1].wait()


def _sc_combine(ysh, routed, sl_flat):
    return pl.kernel(
        _combine_body,
        out_type=jax.ShapeDtypeStruct((T, D), _F32),
        mesh=_SC_MESH,
        compiler_params=pltpu.CompilerParams(needs_layout_passes=False),
        scratch_types=[pltpu.VMEM((2 * _CT,), jnp.int32),
                       pltpu.VMEM((2 * _CT,), jnp.int32),
                       pltpu.VMEM((2 * _CT,), jnp.int32),
                       pltpu.VMEM((2 * _CT,), jnp.int32),
                       pltpu.VMEM((_CT, D), _F32),
                       pltpu.VMEM((_CT, D), _F32),
                       pltpu.VMEM((2 * _CT, D), _F32),
                       pltpu.VMEM((2 * _CT, D), _F32),
                       pltpu.SemaphoreType.DMA,
                       pltpu.SemaphoreType.DMA],
    )(ysh, routed, sl_flat)


def kernel(x, sh_w1, sh_b1, sh_w2, sh_b2, e_w1, e_b1, e_w2, e_b2, r_w, r_b):
    xm = x.reshape(T, D)
    top_idx, top_w = _router(xm, r_w, r_b)
    src_token, slot_w, be_pad, sl_flat = _sc_dispatch(
        top_idx.reshape(-1), top_w.reshape(-1))
    ysh = _shared_mlp(xm, sh_w1, sh_b1, sh_w2, sh_b2)
    xg = _sc_gather(xm, src_token)
    routed = _gmm(xg, e_w1, e_b1, e_w2, e_b2, slot_w, be_pad[:NB])
    out = _sc_combine(ysh, routed, sl_flat)
    return out.reshape(1, T, D)


# speedup vs baseline: 1.0027x; 1.0027x over previous
import functools
import math

import jax
import jax.numpy as jnp
from jax import lax
from jax.experimental import pallas as pl
from jax.experimental.pallas import tpu as pltpu
from jax.experimental.pallas import tpu_sc as plsc

T = 2048
D = 1024
E = 8
HS = 2 * D
HR = 3 * D
TAU = 1.5
BLK = 128
NB = 40
S = NB * BLK
BH = 1024
NBH = HR // BH
BH_S = 1024
NBH_S = HS // BH_S

_F32 = jnp.float32
_PREC = lax.Precision.DEFAULT
_BF16 = jnp.bfloat16


def _gelu(v):
    return 0.5 * v * (1.0 + lax.erf(v * (1.0 / math.sqrt(2.0))))


def _router_body(lg_ref, idx_ref, w_ref):
    logits = lg_ref[...]
    col = lax.broadcasted_iota(jnp.int32, (T, E), 1)
    v1 = jnp.max(logits, axis=-1, keepdims=True)
    i1 = jnp.min(jnp.where(logits == v1, col, E), axis=-1, keepdims=True)
    masked = jnp.where(col == i1, -jnp.inf, logits)
    v2 = jnp.max(masked, axis=-1, keepdims=True)
    i2 = jnp.min(jnp.where(masked == v2, col, E), axis=-1, keepdims=True)
    w1 = 1.0 / (1.0 + jnp.exp(v2 - v1))
    idx_ref[:, 0:1] = i1
    idx_ref[:, 1:2] = i2
    w_ref[:, 0:1] = w1
    w_ref[:, 1:2] = 1.0 - w1


def _router(xm, r_w, r_b):
    logits = (xm @ r_w + r_b) / max(TAU, 1e-6)
    return pl.pallas_call(
        _router_body,
        out_shape=(jax.ShapeDtypeStruct((T, 2), jnp.int32),
                   jax.ShapeDtypeStruct((T, 2), _F32)),
    )(logits)


BT_S = 256
NT_S = T // BT_S


def _shared_body(x_ref, w1a_ref, w1g_ref, b1a_ref, b1g_ref, w2_ref, b2_ref,
                 out_ref):
    x = x_ref[...]
    a = jnp.dot(x, w1a_ref[...], precision=_PREC,
                preferred_element_type=_F32) + b1a_ref[...]
    g = jnp.dot(x, w1g_ref[...], precision=_PREC,
                preferred_element_type=_F32) + b1g_ref[...]
    out_ref[...] = jnp.dot(a * _gelu(g), w2_ref[...], precision=_PREC,
                           preferred_element_type=_F32) + b2_ref[...]


def _shared_mlp(xm, sh_w1, sh_b1, sh_w2, sh_b2):
    b1 = sh_b1.reshape(1, 2 * HS)
    return pl.pallas_call(
        _shared_body,
        grid=(NT_S,),
        in_specs=[
            pl.BlockSpec((BT_S, D), lambda t: (t, 0)),
            pl.BlockSpec((D, HS), lambda t: (0, 0)),
            pl.BlockSpec((D, HS), lambda t: (0, 1)),
            pl.BlockSpec((1, HS), lambda t: (0, 0)),
            pl.BlockSpec((1, HS), lambda t: (0, 1)),
            pl.BlockSpec((HS, D), lambda t: (0, 0)),
            pl.BlockSpec((1, D), lambda t: (0, 0)),
        ],
        out_specs=pl.BlockSpec((BT_S, D), lambda t: (t, 0)),
        out_shape=jax.ShapeDtypeStruct((T, D), _F32),
    )(xm, sh_w1, sh_w1, b1, b1, sh_w2, sh_b2.reshape(1, D))


def _gmm_body(be_ref, xg_ref, w1a_ref, w1g_ref, b1a_ref, b1g_ref, w2_ref,
              b2_ref, sw_ref, out_ref):
    bh = pl.program_id(0)
    b = pl.program_id(1)
    x = xg_ref[...]
    a = jnp.dot(x, w1a_ref[0], precision=_PREC,
                preferred_element_type=_F32) + b1a_ref[0]
    g = jnp.dot(x, w1g_ref[0], precision=_PREC,
                preferred_element_type=_F32) + b1g_ref[0]
    contrib = jnp.dot(a * _gelu(g), w2_ref[0], precision=_PREC,
                      preferred_element_type=_F32)
    rows = pl.ds(b * BLK, BLK)

    @pl.when(bh == 0)
    def _():
        out_ref[rows, :] = contrib

    @pl.when(bh == 1)
    def _():
        out_ref[rows, :] += contrib

    @pl.when(bh == NBH - 1)
    def _():
        out_ref[rows, :] = ((out_ref[rows, :] + contrib + b2_ref[0]) *
                            sw_ref[0, 0][:, None])


def _gmm(xg, e_w1, e_b1, e_w2, e_b2, slot_w, blk_expert):
    grid_spec = pltpu.PrefetchScalarGridSpec(
        num_scalar_prefetch=1,
        grid=(NBH, NB),
        in_specs=[
            pl.BlockSpec((BLK, D), lambda bh, b, be: (b, 0)),
            pl.BlockSpec((1, D, BH), lambda bh, b, be: (be[b], 0, bh)),
            pl.BlockSpec((1, D, BH), lambda bh, b, be: (be[b], 0, bh + NBH)),
            pl.BlockSpec((1, 1, BH), lambda bh, b, be: (be[b], 0, bh)),
            pl.BlockSpec((1, 1, BH), lambda bh, b, be: (be[b], 0, bh + NBH)),
            pl.BlockSpec((1, BH, D), lambda bh, b, be: (be[b], bh, 0)),
            pl.BlockSpec((1, 1, D), lambda bh, b, be: (be[b], 0, 0)),
            pl.BlockSpec((1, 1, BLK), lambda bh, b, be: (b, 0, 0)),
        ],
        out_specs=pl.BlockSpec((S, D), lambda bh, b, be: (0, 0)),
    )
    return pl.pallas_call(
        _gmm_body,
        grid_spec=grid_spec,
        out_shape=jax.ShapeDtypeStruct((S, D), _F32),
    )(blk_expert, xg, e_w1, e_w1, e_b1.reshape(E, 1, 2 * HR),
      e_b1.reshape(E, 1, 2 * HR), e_w2, e_b2.reshape(E, 1, D),
      slot_w.reshape(NB, 1, BLK))


_NC, _NS, _L = 2, 16, 16
_NW = _NC * _NS
_NSTEP = (2 * T) // _L
_SINIT = S // _L
_NBPAD = 48

_SC_MESH = plsc.VectorSubcoreMesh(core_axis_name="c", subcore_axis_name="s")


def _wid():
    return lax.axis_index("s") * _NC + lax.axis_index("c")


def _vgather(vec, idx):
    return vec.at[idx].get(mode="promise_in_bounds")


def _dispatch_body(ti_hbm, tw_hbm, st_hbm, sw_hbm, be_hbm, sl_hbm,
                   ids_v, w_v, st_v, sw_v, be_v, sl_v):
    @pl.when(_wid() == 0)
    def _():
        pltpu.sync_copy(ti_hbm, ids_v)
        pltpu.sync_copy(tw_hbm, w_v)
        lanes = lax.iota(jnp.int32, _L)

        def count_step(i, counts):
            idv = ids_v[pl.ds(i * _L, _L)]
            for e in range(E):
                c = jnp.sum(jnp.where(idv == e, 1, 0))
                counts = counts + jnp.where(lanes == e, c, 0)
            return counts

        counts = lax.fori_loop(0, _NSTEP, count_step,
                               jnp.zeros((_L,), jnp.int32))
        blocks = (counts + (BLK - 1)) >> 7
        cumb = plsc.cumsum(blocks)
        row_off = (cumb - blocks) * BLK

        for ch in range(_NBPAD // _L):
            jb = lanes + ch * _L
            be = jnp.zeros((_L,), jnp.int32)
            for e in range(E - 1):
                ce = jnp.sum(jnp.where(lanes == e, cumb, 0))
                be = be + jnp.where(jb >= ce, 1, 0)
            be_v[pl.ds(ch * _L, _L)] = be

        def zero_step(i, carry):
            st_v[pl.ds(i * _L, _L)] = jnp.zeros((_L,), jnp.int32)
            sw_v[pl.ds(i * _L, _L)] = jnp.zeros((_L,), _F32)
            return carry

        lax.fori_loop(0, _SINIT, zero_step, 0)

        def scat_step(i, fill):
            idv = ids_v[pl.ds(i * _L, _L)]
            rank = jnp.zeros((_L,), jnp.int32)
            newfill = fill
            for e in range(E):
                m = idv == e
                mi = jnp.where(m, 1, 0)
                cs = plsc.cumsum(mi)
                rank = jnp.where(m, cs - mi, rank)
                c = jnp.sum(jnp.where(lanes == _L - 1, cs, 0))
                newfill = newfill + jnp.where(lanes == e, c, 0)
            slot = _vgather(row_off + fill, idv) + rank
            tok = (lanes + i * _L) >> 1
            plsc.store_scatter(st_v, [slot], tok)
            plsc.store_scatter(sw_v, [slot], w_v[pl.ds(i * _L, _L)])
            sl_v[pl.ds(i * _L, _L)] = slot
            return newfill

        lax.fori_loop(0, _NSTEP, scat_step, jnp.zeros((_L,), jnp.int32))
        pltpu.sync_copy(st_v, st_hbm)
        pltpu.sync_copy(sw_v, sw_hbm)
        pltpu.sync_copy(be_v, be_hbm)
        pltpu.sync_copy(sl_v, sl_hbm)


def _sc_dispatch(ti_flat, tw_flat):
    return pl.kernel(
        _dispatch_body,
        out_type=(jax.ShapeDtypeStruct((S,), jnp.int32),
                  jax.ShapeDtypeStruct((S,), _F32),
                  jax.ShapeDtypeStruct((_NBPAD,), jnp.int32),
                  jax.ShapeDtypeStruct((2 * T,), jnp.int32)),
        mesh=_SC_MESH,
        compiler_params=pltpu.CompilerParams(needs_layout_passes=False),
        scratch_types=[pltpu.VMEM((2 * T,), jnp.int32),
                       pltpu.VMEM((2 * T,), _F32),
                       pltpu.VMEM((S,), jnp.int32),
                       pltpu.VMEM((S,), _F32),
                       pltpu.VMEM((_NBPAD,), jnp.int32),
                       pltpu.VMEM((2 * T,), jnp.int32)],
    )(ti_flat, tw_flat)


_RPW = S // _NW
_GCH = 40
_NGCH = _RPW // _GCH


def _gather_body(x_hbm, idx_hbm, out_hbm, i0, i1, i2, i3, r0, r1,
                 gsem, ssem):
    base = _wid() * _RPW
    idx = [i0, i1, i2, i3]
    rows = [r0, r1]
    ic = [pltpu.async_copy(idx_hbm.at[pl.ds(base + ch * _GCH, _GCH)],
                           idx[ch], ssem) for ch in range(_NGCH)]
    for c in ic:
        c.wait()

    def _fire(ch):
        return pltpu.async_copy(x_hbm.at[idx[ch]], rows[ch % 2], gsem)

    def _store(ch):
        return pltpu.async_copy(rows[ch % 2],
                                out_hbm.at[pl.ds(base + ch * _GCH, _GCH)],
                                ssem)

    g = {0: _fire(0)}
    st = {}
    for ch in range(_NGCH):
        if ch + 1 < _NGCH:
            if ch - 1 >= 0:
                st[ch - 1].wait()
            g[ch + 1] = _fire(ch + 1)
        g[ch].wait()
        st[ch] = _store(ch)
    st[_NGCH - 2].wait()
    st[_NGCH - 1].wait()


def _sc_gather(xm, src_token):
    return pl.kernel(
        _gather_body,
        out_type=jax.ShapeDtypeStruct((S, D), _F32),
        mesh=_SC_MESH,
        compiler_params=pltpu.CompilerParams(needs_layout_passes=False),
        scratch_types=[pltpu.VMEM((_GCH,), jnp.int32),
                       pltpu.VMEM((_GCH,), jnp.int32),
                       pltpu.VMEM((_GCH,), jnp.int32),
                       pltpu.VMEM((_GCH,), jnp.int32),
                       pltpu.VMEM((_GCH, D), _F32),
                       pltpu.VMEM((_GCH, D), _F32),
                       pltpu.SemaphoreType.DMA,
                       pltpu.SemaphoreType.DMA],
    )(xm, src_token)


_TPW = T // _NW
_CT = 16
_NCT = _TPW // _CT


def _combine_body(ysh_hbm, routed_hbm, sl_hbm, out_hbm,
                  i0, i1, i2, i3, y0, y1, r0, r1, gsem, ssem):
    tb = _wid() * _TPW
    idx = [i0, i1, i2, i3]
    yb = [y0, y1]
    rb = [r0, r1]
    ic = [pltpu.async_copy(sl_hbm.at[pl.ds(2 * (tb + ch * _CT), 2 * _CT)],
                           idx[ch], ssem) for ch in range(_NCT)]
    for c in ic:
        c.wait()

    def _fire(ch):
        b = ch % 2
        return (pltpu.async_copy(ysh_hbm.at[pl.ds(tb + ch * _CT, _CT)],
                                 yb[b], gsem),
                pltpu.async_copy(routed_hbm.at[idx[ch]], rb[b], gsem))

    def _store(ch):
        return pltpu.async_copy(yb[ch % 2],
                                out_hbm.at[pl.ds(tb + ch * _CT, _CT)], ssem)

    g = {0: _fire(0)}
    st = {}
    for ch in range(_NCT):
        b = ch % 2
        if ch + 1 < _NCT:
            if ch - 1 >= 0:
                st[ch - 1].wait()
            g[ch + 1] = _fire(ch + 1)
        g[ch][0].wait()
        g[ch][1].wait()

        def tok_step(tt, carry):
            def col_step(cc, carry2):
                c0 = cc * _L
                yb[b][tt, pl.ds(c0, _L)] = (yb[b][tt, pl.ds(c0, _L)]
                                            + rb[b][2 * tt, pl.ds(c0, _L)]
                                            + rb[b][2 * tt + 1, pl.ds(c0, _L)])
                return carry2

            lax.fori_loop(0, D // _L, col_step, 0)
            return carry

        lax.fori_loop(0, _CT, tok_step, 0)
        st[ch] = _store(ch)
    st[_NCT - 2].wait()
    st[_NCT - 1].wait()


def _sc_combine(ysh, routed, sl_flat):
    return pl.kernel(
        _combine_body,
        out_type=jax.ShapeDtypeStruct((T, D), _F32),
        mesh=_SC_MESH,
        compiler_params=pltpu.CompilerParams(needs_layout_passes=False),
        scratch_types=[pltpu.VMEM((2 * _CT,), jnp.int32),
                       pltpu.VMEM((2 * _CT,), jnp.int32),
                       pltpu.VMEM((2 * _CT,), jnp.int32),
                       pltpu.VMEM((2 * _CT,), jnp.int32),
                       pltpu.VMEM((_CT, D), _F32),
                       pltpu.VMEM((_CT, D), _F32),
                       pltpu.VMEM((2 * _CT, D), _F32),
                       pltpu.VMEM((2 * _CT, D), _F32),
                       pltpu.SemaphoreType.DMA,
                       pltpu.SemaphoreType.DMA],
    )(ysh, routed, sl_flat)


def kernel(x, sh_w1, sh_b1, sh_w2, sh_b2, e_w1, e_b1, e_w2, e_b2, r_w, r_b):
    xm = x.reshape(T, D)
    top_idx, top_w = _router(xm, r_w, r_b)
    src_token, slot_w, be_pad, sl_flat = _sc_dispatch(
        top_idx.reshape(-1), top_w.reshape(-1))
    ysh = _shared_mlp(xm, sh_w1, sh_b1, sh_w2, sh_b2)
    xg = _sc_gather(xm, src_token)
    routed = _gmm(xg, e_w1, e_b1, e_w2, e_b2, slot_w, be_pad[:NB])
    out = _sc_combine(ysh, routed, sl_flat)
    return out.reshape(1, T, D)
